# Initial kernel scaffold; baseline (speedup 1.0000x reference)
import jax
import jax.numpy as jnp
from jax.experimental import pallas as pl


def kernel(x, edge_index, W1, a_src1, a_dst1, b1, W2, a_src2, a_dst2, b2):
    raise NotImplementedError("write your pallas kernel here")


# speedup vs baseline: 34.2959x; 34.2959x over previous
import functools

import jax
import jax.numpy as jnp
from jax import lax
from jax.experimental import pallas as pl
from jax.experimental.pallas import tpu as pltpu
from jax.experimental.pallas import tpu_sc as plsc

_N = 10000
_E = 320000
_DF = 128
_DH = 16
_NC = 7

_NCORE = 2
_NSUB = 16
_NW = _NCORE * _NSUB
_EPW = _E // _NW
_CHUNK = 80
_NCHUNK = _EPW // _CHUNK
_NGRP = _CHUNK // 16
_NP = 10240
_RPT = _NP // _NSUB
_ZROWS = 128
_NZ = _RPT // _ZROWS
_DACC = 32


def _dense1_body(x_ref, w_ref, asrc_ref, adst_ref, h_ref, es_ref, ed_ref):
    h = jnp.dot(x_ref[...], w_ref[...], preferred_element_type=jnp.float32)
    h_ref[...] = h
    es_ref[...] = jnp.sum(h * asrc_ref[...], axis=1, keepdims=True)
    ed_ref[...] = jnp.sum(h * adst_ref[...], axis=1, keepdims=True)


_dense1 = pl.pallas_call(
    _dense1_body,
    out_shape=(
        jax.ShapeDtypeStruct((_N, _DH), jnp.float32),
        jax.ShapeDtypeStruct((_N, 1), jnp.float32),
        jax.ShapeDtypeStruct((_N, 1), jnp.float32),
    ),
)


def _edge_body(src_hbm, dst_hbm, es_hbm, ed_hbm, tab_hbm, acc_hbm,
               es_v, ed_v, sidx, didx, rows, wrows, zbuf, acc_sh,
               sem):
    c = lax.axis_index("c")
    s = lax.axis_index("s")
    wid = s * _NCORE + c

    pltpu.sync_copy(es_hbm, es_v)
    pltpu.sync_copy(ed_hbm, ed_v)

    z16 = jnp.zeros((16,), jnp.float32)

    def zero_zbuf(r, carry):
        zbuf[r, pl.ds(0, 16)] = z16
        zbuf[r, pl.ds(16, 16)] = z16
        return carry

    lax.fori_loop(0, _ZROWS, zero_zbuf, 0)

    def zero_wrows_pad(r, carry):
        wrows[r, pl.ds(16, 16)] = z16
        return carry

    lax.fori_loop(0, _CHUNK, zero_wrows_pad, 0)

    base_row = s * _RPT
    for k in range(_NZ):
        pltpu.sync_copy(zbuf, acc_sh.at[pl.ds(base_row + k * _ZROWS, _ZROWS)])
    plsc.subcore_barrier()

    lane = lax.iota(jnp.int32, 16)
    col16 = jnp.full((16,), _DH, jnp.int32)

    def chunk(i, carry):
        ebase = wid * _EPW + i * _CHUNK
        pltpu.sync_copy(src_hbm.at[pl.ds(ebase, _CHUNK)], sidx)
        pltpu.sync_copy(dst_hbm.at[pl.ds(ebase, _CHUNK)], didx)
        pltpu.async_copy(tab_hbm.at[sidx], rows, sem).wait()
        for g in range(_NGRP):
            off = g * 16
            sv = sidx[pl.ds(off, 16)]
            dv = didx[pl.ds(off, 16)]
            esv = plsc.load_gather(es_v, [sv])
            edv = plsc.load_gather(ed_v, [dv])
            a = esv + edv
            a = jnp.where(a > 0.0, a, a * jnp.float32(0.2))
            w = jnp.exp(a)
            plsc.store_scatter(wrows, [lane + off, col16], w)
            for j in range(16):
                wj = w[j]
                wrows[off + j, pl.ds(0, _DH)] = rows[off + j] * wj
        pltpu.sync_copy(wrows, acc_sh.at[didx], add=True)
        return carry

    lax.fori_loop(0, _NCHUNK, chunk, 0)

    plsc.subcore_barrier()
    out_base = c * _NP + base_row
    for k in range(_NZ):
        pltpu.sync_copy(acc_sh.at[pl.ds(base_row + k * _ZROWS, _ZROWS)],
                        acc_hbm.at[pl.ds(out_base + k * _ZROWS, _ZROWS)])


_edge_pass = pl.kernel(
    _edge_body,
    mesh=plsc.VectorSubcoreMesh(core_axis_name="c", subcore_axis_name="s"),
    out_type=jax.ShapeDtypeStruct((_NCORE * _NP, _DACC), jnp.float32),
    scratch_types=[
        pltpu.VMEM((_N,), jnp.float32),
        pltpu.VMEM((_N,), jnp.float32),
        pltpu.VMEM((_CHUNK,), jnp.int32),
        pltpu.VMEM((_CHUNK,), jnp.int32),
        pltpu.VMEM((_CHUNK, _DH), jnp.float32),
        pltpu.VMEM((_CHUNK, _DACC), jnp.float32),
        pltpu.VMEM((_ZROWS, _DACC), jnp.float32),
        pltpu.VMEM_SHARED((_NP, _DACC), jnp.float32),
        pltpu.SemaphoreType.DMA,
    ],
    compiler_params=pltpu.CompilerParams(needs_layout_passes=False,
                                         use_tc_tiling_on_sc=False),
)


def _combine1_body(acc_ref, h_ref, es_ref, ed_ref, b1_ref, w2_ref,
                   asrc2_ref, adst2_ref, h2_ref, es2_ref, ed2_ref):
    acc = acc_ref[0] + acc_ref[1]
    num = acc[:, 0:_DH]
    den = acc[:, _DH:_DH + 1]
    ew = es_ref[...] + ed_ref[...]
    ew = jnp.where(ew > 0.0, ew, ew * jnp.float32(0.2))
    wself = jnp.exp(ew)
    num = num + wself * h_ref[...]
    den = den + wself
    out1 = num / den + b1_ref[...]
    out1 = jnp.where(out1 > 0.0, out1, jnp.exp(out1) - 1.0)
    h2 = jnp.dot(out1, w2_ref[...], preferred_element_type=jnp.float32)
    h2_ref[...] = h2
    es2_ref[...] = jnp.sum(h2 * asrc2_ref[...], axis=1, keepdims=True)
    ed2_ref[...] = jnp.sum(h2 * adst2_ref[...], axis=1, keepdims=True)


_combine1 = pl.pallas_call(
    _combine1_body,
    out_shape=(
        jax.ShapeDtypeStruct((_N, _DH), jnp.float32),
        jax.ShapeDtypeStruct((_N, 1), jnp.float32),
        jax.ShapeDtypeStruct((_N, 1), jnp.float32),
    ),
)


def _final_body(acc_ref, h2_ref, es2_ref, ed2_ref, b2_ref, out_ref):
    acc = acc_ref[0] + acc_ref[1]
    num = acc[:, 0:_DH]
    den = acc[:, _DH:_DH + 1]
    ew = es2_ref[...] + ed2_ref[...]
    ew = jnp.where(ew > 0.0, ew, ew * jnp.float32(0.2))
    wself = jnp.exp(ew)
    num = num + wself * h2_ref[...]
    den = den + wself
    o = num / den + b2_ref[...]
    o7 = o[:, 0:_NC]
    m = jnp.max(o7, axis=1, keepdims=True)
    z = o7 - m
    lse = jnp.log(jnp.sum(jnp.exp(z), axis=1, keepdims=True))
    out_ref[...] = z - lse


_final = pl.pallas_call(
    _final_body,
    out_shape=jax.ShapeDtypeStruct((_N, _NC), jnp.float32),
)


def kernel(x, edge_index, W1, a_src1, a_dst1, b1, W2, a_src2, a_dst2, b2):
    src = edge_index[0]
    dst = edge_index[1]

    h1, es1, ed1 = _dense1(x, W1, a_src1.reshape(1, _DH),
                           a_dst1.reshape(1, _DH))

    acc1 = _edge_pass(src, dst, es1.reshape(_N), ed1.reshape(_N), h1)
    acc1 = acc1.reshape(_NCORE, _NP, _DACC)[:, :_N, :]

    pad = _DH - _NC
    W2p = jnp.pad(W2, ((0, 0), (0, pad)))
    asrc2p = jnp.pad(a_src2, (0, pad)).reshape(1, _DH)
    adst2p = jnp.pad(a_dst2, (0, pad)).reshape(1, _DH)
    b2p = jnp.pad(b2, (0, pad)).reshape(1, _DH)

    h2, es2, ed2 = _combine1(acc1, h1, es1, ed1, b1.reshape(1, _DH),
                             W2p, asrc2p, adst2p)

    acc2 = _edge_pass(src, dst, es2.reshape(_N), ed2.reshape(_N), h2)
    acc2 = acc2.reshape(_NCORE, _NP, _DACC)[:, :_N, :]

    return _final(acc2, h2, es2, ed2, b2p)


# speedup vs baseline: 71.9874x; 2.0990x over previous
import functools

import jax
import jax.numpy as jnp
from jax import lax
from jax.experimental import pallas as pl
from jax.experimental.pallas import tpu as pltpu
from jax.experimental.pallas import tpu_sc as plsc

_N = 10000
_E = 320000
_DF = 128
_DH = 16
_NC = 7

_NCORE = 2
_NSUB = 16
_NW = _NCORE * _NSUB
_EPW = _E // _NW
_CHUNK = 80
_NCHUNK = _EPW // _CHUNK
_NGRP = _CHUNK // 16
_CROWS = 128
_NP = 10240
_RPT = _NP // _NSUB
_ZROWS = 128
_NZ = _RPT // _ZROWS
_DACC = 32


def _dense1_body(x_ref, w_ref, asrc_ref, adst_ref, h_ref, es_ref, ed_ref):
    h = jnp.dot(x_ref[...], w_ref[...], preferred_element_type=jnp.float32)
    h_ref[...] = h
    es_ref[...] = jnp.sum(h * asrc_ref[...], axis=1, keepdims=True)
    ed_ref[...] = jnp.sum(h * adst_ref[...], axis=1, keepdims=True)


_dense1 = pl.pallas_call(
    _dense1_body,
    out_shape=(
        jax.ShapeDtypeStruct((_N, _DH), jnp.float32),
        jax.ShapeDtypeStruct((_N, 1), jnp.float32),
        jax.ShapeDtypeStruct((_N, 1), jnp.float32),
    ),
)


def _edge_body(src_hbm, dst_hbm, es_hbm, ed_hbm, tab_hbm, acc_hbm,
               es_v, ed_v, sidx2, didx2, dbuf0, dbuf1, rows0, rows1,
               wrows0, wrows1, zbuf, acc_sh, gs0, gs1):
    c = lax.axis_index("c")
    s = lax.axis_index("s")
    wid = s * _NCORE + c

    pltpu.sync_copy(es_hbm, es_v)
    pltpu.sync_copy(ed_hbm, ed_v)
    pltpu.sync_copy(src_hbm.at[pl.ds(wid * _CROWS, _CROWS)], sidx2)
    pltpu.sync_copy(dst_hbm.at[pl.ds(wid * _CROWS, _CROWS)], didx2)

    z16 = jnp.zeros((16,), jnp.float32)

    def zero_zbuf(r, carry):
        zbuf[r, pl.ds(0, 16)] = z16
        zbuf[r, pl.ds(16, 16)] = z16
        return carry

    lax.fori_loop(0, _ZROWS, zero_zbuf, 0)

    def zero_wrows(r, carry):
        wrows0[r, pl.ds(16, 16)] = z16
        wrows1[r, pl.ds(16, 16)] = z16
        return carry

    lax.fori_loop(0, _CHUNK, zero_wrows, 0)

    base_row = s * _RPT
    for k in range(_NZ):
        pltpu.sync_copy(zbuf, acc_sh.at[pl.ds(base_row + k * _ZROWS, _ZROWS)])

    pltpu.make_async_copy(tab_hbm.at[sidx2.at[0]], rows0, gs0).start()
    pltpu.make_async_copy(tab_hbm.at[sidx2.at[1]], rows1, gs1).start()
    plsc.subcore_barrier()

    lane = lax.iota(jnp.int32, 16)
    col16 = jnp.full((16,), _DH, jnp.int32)

    def compute(i, rows, wrows, dbuf):
        for g in range(_NGRP):
            off = g * 16
            sv = sidx2[i, pl.ds(off, 16)]
            dv = didx2[i, pl.ds(off, 16)]
            dbuf[pl.ds(off, 16)] = dv
            esv = plsc.load_gather(es_v, [sv])
            edv = plsc.load_gather(ed_v, [dv])
            a = esv + edv
            a = jnp.where(a > 0.0, a, a * jnp.float32(0.2))
            w = jnp.exp(a)
            plsc.store_scatter(wrows, [lane + off, col16], w)
            for j in range(16):
                wj = w[j]
                wrows[off + j, pl.ds(0, _DH)] = rows[off + j] * wj

    def half(i, rows, wrows, dbuf, gsem):
        pltpu.make_async_copy(tab_hbm.at[sidx2.at[i]], rows, gsem).wait()
        compute(i, rows, wrows, dbuf)
        pltpu.make_async_copy(tab_hbm.at[sidx2.at[i + 2]], rows, gsem).start()
        pltpu.sync_copy(wrows, acc_sh.at[dbuf], add=True)

    def pair(k, carry):
        half(2 * k, rows0, wrows0, dbuf0, gs0)
        half(2 * k + 1, rows1, wrows1, dbuf1, gs1)
        return carry

    lax.fori_loop(0, (_NCHUNK - 1) // 2, pair, 0)

    last = _NCHUNK - 1
    pltpu.make_async_copy(tab_hbm.at[sidx2.at[last]], rows0, gs0).wait()
    compute(last, rows0, wrows0, dbuf0)
    pltpu.sync_copy(wrows0, acc_sh.at[dbuf0], add=True)
    pltpu.make_async_copy(tab_hbm.at[sidx2.at[last + 1]], rows1, gs1).wait()

    plsc.subcore_barrier()
    out_base = c * _NP + base_row
    for k in range(_NZ):
        pltpu.sync_copy(acc_sh.at[pl.ds(base_row + k * _ZROWS, _ZROWS)],
                        acc_hbm.at[pl.ds(out_base + k * _ZROWS, _ZROWS)])


_edge_pass = pl.kernel(
    _edge_body,
    mesh=plsc.VectorSubcoreMesh(core_axis_name="c", subcore_axis_name="s"),
    out_type=jax.ShapeDtypeStruct((_NCORE * _NP, _DACC), jnp.float32),
    scratch_types=[
        pltpu.VMEM((_N,), jnp.float32),
        pltpu.VMEM((_N,), jnp.float32),
        pltpu.VMEM((_CROWS, _CHUNK), jnp.int32),
        pltpu.VMEM((_CROWS, _CHUNK), jnp.int32),
        pltpu.VMEM((_CHUNK,), jnp.int32),
        pltpu.VMEM((_CHUNK,), jnp.int32),
        pltpu.VMEM((_CHUNK, _DH), jnp.float32),
        pltpu.VMEM((_CHUNK, _DH), jnp.float32),
        pltpu.VMEM((_CHUNK, _DACC), jnp.float32),
        pltpu.VMEM((_CHUNK, _DACC), jnp.float32),
        pltpu.VMEM((_ZROWS, _DACC), jnp.float32),
        pltpu.VMEM_SHARED((_NP, _DACC), jnp.float32),
        pltpu.SemaphoreType.DMA,
        pltpu.SemaphoreType.DMA,
    ],
    compiler_params=pltpu.CompilerParams(needs_layout_passes=False,
                                         use_tc_tiling_on_sc=False),
)


def _combine1_body(acc_ref, h_ref, es_ref, ed_ref, b1_ref, w2_ref,
                   asrc2_ref, adst2_ref, h2_ref, es2_ref, ed2_ref):
    acc = acc_ref[0] + acc_ref[1]
    num = acc[:, 0:_DH]
    den = acc[:, _DH:_DH + 1]
    ew = es_ref[...] + ed_ref[...]
    ew = jnp.where(ew > 0.0, ew, ew * jnp.float32(0.2))
    wself = jnp.exp(ew)
    num = num + wself * h_ref[...]
    den = den + wself
    out1 = num / den + b1_ref[...]
    out1 = jnp.where(out1 > 0.0, out1, jnp.exp(out1) - 1.0)
    h2 = jnp.dot(out1, w2_ref[...], preferred_element_type=jnp.float32)
    h2_ref[...] = h2
    es2_ref[...] = jnp.sum(h2 * asrc2_ref[...], axis=1, keepdims=True)
    ed2_ref[...] = jnp.sum(h2 * adst2_ref[...], axis=1, keepdims=True)


_combine1 = pl.pallas_call(
    _combine1_body,
    out_shape=(
        jax.ShapeDtypeStruct((_N, _DH), jnp.float32),
        jax.ShapeDtypeStruct((_N, 1), jnp.float32),
        jax.ShapeDtypeStruct((_N, 1), jnp.float32),
    ),
)


def _final_body(acc_ref, h2_ref, es2_ref, ed2_ref, b2_ref, out_ref):
    acc = acc_ref[0] + acc_ref[1]
    num = acc[:, 0:_DH]
    den = acc[:, _DH:_DH + 1]
    ew = es2_ref[...] + ed2_ref[...]
    ew = jnp.where(ew > 0.0, ew, ew * jnp.float32(0.2))
    wself = jnp.exp(ew)
    num = num + wself * h2_ref[...]
    den = den + wself
    o = num / den + b2_ref[...]
    o7 = o[:, 0:_NC]
    m = jnp.max(o7, axis=1, keepdims=True)
    z = o7 - m
    lse = jnp.log(jnp.sum(jnp.exp(z), axis=1, keepdims=True))
    out_ref[...] = z - lse


_final = pl.pallas_call(
    _final_body,
    out_shape=jax.ShapeDtypeStruct((_N, _NC), jnp.float32),
)


def kernel(x, edge_index, W1, a_src1, a_dst1, b1, W2, a_src2, a_dst2, b2):
    sd = edge_index.reshape(2, _NW, _NCHUNK, _CHUNK)
    sd = jnp.pad(sd, ((0, 0), (0, 0), (0, _CROWS - _NCHUNK), (0, 0)))
    src = sd[0].reshape(_NW * _CROWS, _CHUNK)
    dst = sd[1].reshape(_NW * _CROWS, _CHUNK)

    h1, es1, ed1 = _dense1(x, W1, a_src1.reshape(1, _DH),
                           a_dst1.reshape(1, _DH))

    acc1 = _edge_pass(src, dst, es1.reshape(_N), ed1.reshape(_N), h1)
    acc1 = acc1.reshape(_NCORE, _NP, _DACC)[:, :_N, :]

    pad = _DH - _NC
    W2p = jnp.pad(W2, ((0, 0), (0, pad)))
    asrc2p = jnp.pad(a_src2, (0, pad)).reshape(1, _DH)
    adst2p = jnp.pad(a_dst2, (0, pad)).reshape(1, _DH)
    b2p = jnp.pad(b2, (0, pad)).reshape(1, _DH)

    h2, es2, ed2 = _combine1(acc1, h1, es1, ed1, b1.reshape(1, _DH),
                             W2p, asrc2p, adst2p)

    acc2 = _edge_pass(src, dst, es2.reshape(_N), ed2.reshape(_N), h2)
    acc2 = acc2.reshape(_NCORE, _NP, _DACC)[:, :_N, :]

    return _final(acc2, h2, es2, ed2, b2p)


# speedup vs baseline: 73.5380x; 1.0215x over previous
import functools

import jax
import jax.numpy as jnp
from jax import lax
from jax.experimental import pallas as pl
from jax.experimental.pallas import tpu as pltpu
from jax.experimental.pallas import tpu_sc as plsc

_N = 10000
_E = 320000
_DF = 128
_DH = 16
_NC = 7

_NCORE = 2
_NSUB = 16
_NW = _NCORE * _NSUB
_EPW = _E // _NW
_CHUNK = 80
_NCHUNK = _EPW // _CHUNK
_NGRP = _CHUNK // 16
_CROWS = 128
_NP = 10240
_RPT = _NP // _NSUB
_ZROWS = 128
_NZ = _RPT // _ZROWS
_DACC = 32


def _dense1_body(x_ref, w_ref, asrc_ref, adst_ref, h_ref, es_ref, ed_ref):
    h = jnp.dot(x_ref[...], w_ref[...], preferred_element_type=jnp.float32)
    h_ref[...] = h
    es_ref[...] = jnp.sum(h * asrc_ref[...], axis=1, keepdims=True)
    ed_ref[...] = jnp.sum(h * adst_ref[...], axis=1, keepdims=True)


_dense1 = pl.pallas_call(
    _dense1_body,
    out_shape=(
        jax.ShapeDtypeStruct((_N, _DH), jnp.float32),
        jax.ShapeDtypeStruct((_N, 1), jnp.float32),
        jax.ShapeDtypeStruct((_N, 1), jnp.float32),
    ),
)


def _make_edge_body(dacc, with_col16):
    def body(src_hbm, dst_hbm, es_hbm, ed_hbm, tab_hbm, acc_hbm,
             es_v, ed_v, sidx2, didx2, dbuf0, dbuf1, rows0, rows1,
             wrows0, wrows1, zbuf, acc_sh, gs0, gs1):
        c = lax.axis_index("c")
        s = lax.axis_index("s")
        wid = s * _NCORE + c

        pltpu.sync_copy(es_hbm, es_v)
        pltpu.sync_copy(ed_hbm, ed_v)
        pltpu.sync_copy(src_hbm.at[pl.ds(wid * _CROWS, _CROWS)], sidx2)
        pltpu.sync_copy(dst_hbm.at[pl.ds(wid * _CROWS, _CROWS)], didx2)

        z16 = jnp.zeros((16,), jnp.float32)

        def zero_zbuf(r, carry):
            for q in range(dacc // 16):
                zbuf[r, pl.ds(16 * q, 16)] = z16
            return carry

        lax.fori_loop(0, _ZROWS, zero_zbuf, 0)

        if dacc > 16:
            def zero_wrows(r, carry):
                wrows0[r, pl.ds(8, 16)] = z16
                wrows1[r, pl.ds(8, 16)] = z16
                return carry

            lax.fori_loop(0, _CHUNK, zero_wrows, 0)

        base_row = s * _RPT
        for k in range(_NZ):
            pltpu.sync_copy(zbuf,
                            acc_sh.at[pl.ds(base_row + k * _ZROWS, _ZROWS)])

        pltpu.make_async_copy(tab_hbm.at[sidx2.at[0]], rows0, gs0).start()
        pltpu.make_async_copy(tab_hbm.at[sidx2.at[1]], rows1, gs1).start()
        plsc.subcore_barrier()

        lane = lax.iota(jnp.int32, 16)
        col16 = jnp.full((16,), 16, jnp.int32)

        def compute(i, rows, wrows, dbuf):
            for g in range(_NGRP):
                off = g * 16
                sv = sidx2[i, pl.ds(off, 16)]
                dv = didx2[i, pl.ds(off, 16)]
                dbuf[pl.ds(off, 16)] = dv
                esv = plsc.load_gather(es_v, [sv])
                edv = plsc.load_gather(ed_v, [dv])
                a = esv + edv
                a = jnp.where(a > 0.0, a, a * jnp.float32(0.2))
                w = jnp.exp(a)
                if with_col16:
                    plsc.store_scatter(wrows, [lane + off, col16], w)
                for j in range(16):
                    wj = w[j]
                    wrows[off + j, pl.ds(0, _DH)] = rows[off + j] * wj

        def half(i, rows, wrows, dbuf, gsem):
            pltpu.make_async_copy(tab_hbm.at[sidx2.at[i]], rows, gsem).wait()
            compute(i, rows, wrows, dbuf)
            pltpu.make_async_copy(tab_hbm.at[sidx2.at[i + 2]], rows,
                                  gsem).start()
            pltpu.sync_copy(wrows, acc_sh.at[dbuf], add=True)

        def pair(k, carry):
            half(2 * k, rows0, wrows0, dbuf0, gs0)
            half(2 * k + 1, rows1, wrows1, dbuf1, gs1)
            return carry

        lax.fori_loop(0, (_NCHUNK - 1) // 2, pair, 0)

        last = _NCHUNK - 1
        pltpu.make_async_copy(tab_hbm.at[sidx2.at[last]], rows0, gs0).wait()
        compute(last, rows0, wrows0, dbuf0)
        pltpu.sync_copy(wrows0, acc_sh.at[dbuf0], add=True)
        pltpu.make_async_copy(tab_hbm.at[sidx2.at[last + 1]], rows1,
                              gs1).wait()

        plsc.subcore_barrier()
        out_base = c * _NP + base_row
        for k in range(_NZ):
            pltpu.sync_copy(acc_sh.at[pl.ds(base_row + k * _ZROWS, _ZROWS)],
                            acc_hbm.at[pl.ds(out_base + k * _ZROWS, _ZROWS)])

    return body


def _make_edge_pass(dacc, with_col16):
    return pl.kernel(
        _make_edge_body(dacc, with_col16),
        mesh=plsc.VectorSubcoreMesh(core_axis_name="c", subcore_axis_name="s"),
        out_type=jax.ShapeDtypeStruct((_NCORE * _NP, dacc), jnp.float32),
        scratch_types=[
            pltpu.VMEM((_N,), jnp.float32),
            pltpu.VMEM((_N,), jnp.float32),
            pltpu.VMEM((_CROWS, _CHUNK), jnp.int32),
            pltpu.VMEM((_CROWS, _CHUNK), jnp.int32),
            pltpu.VMEM((_CHUNK,), jnp.int32),
            pltpu.VMEM((_CHUNK,), jnp.int32),
            pltpu.VMEM((_CHUNK, _DH), jnp.float32),
            pltpu.VMEM((_CHUNK, _DH), jnp.float32),
            pltpu.VMEM((_CHUNK, dacc), jnp.float32),
            pltpu.VMEM((_CHUNK, dacc), jnp.float32),
            pltpu.VMEM((_ZROWS, dacc), jnp.float32),
            pltpu.VMEM_SHARED((_NP, dacc), jnp.float32),
            pltpu.SemaphoreType.DMA,
            pltpu.SemaphoreType.DMA,
        ],
        compiler_params=pltpu.CompilerParams(needs_layout_passes=False,
                                             use_tc_tiling_on_sc=False),
    )


_DACC1 = 24
_DACC2 = 16
_edge_pass1 = _make_edge_pass(_DACC1, True)
_edge_pass2 = _make_edge_pass(_DACC2, False)


def _combine1_body(acc_ref, h_ref, es_ref, ed_ref, b1_ref, w2_ref,
                   asrc2_ref, adst2_ref, h2_ref, es2_ref, ed2_ref):
    acc = acc_ref[0] + acc_ref[1]
    num = acc[:, 0:_DH]
    den = acc[:, _DH:_DH + 1]
    ew = es_ref[...] + ed_ref[...]
    ew = jnp.where(ew > 0.0, ew, ew * jnp.float32(0.2))
    wself = jnp.exp(ew)
    num = num + wself * h_ref[...]
    den = den + wself
    out1 = num / den + b1_ref[...]
    out1 = jnp.where(out1 > 0.0, out1, jnp.exp(out1) - 1.0)
    h2 = jnp.dot(out1, w2_ref[...], preferred_element_type=jnp.float32)
    one7 = (lax.broadcasted_iota(jnp.int32, (1, _DH), 1)
            == _NC).astype(jnp.float32)
    h2_ref[...] = h2 + one7
    es2_ref[...] = jnp.sum(h2 * asrc2_ref[...], axis=1, keepdims=True)
    ed2_ref[...] = jnp.sum(h2 * adst2_ref[...], axis=1, keepdims=True)


_combine1 = pl.pallas_call(
    _combine1_body,
    out_shape=(
        jax.ShapeDtypeStruct((_N, _DH), jnp.float32),
        jax.ShapeDtypeStruct((_N, 1), jnp.float32),
        jax.ShapeDtypeStruct((_N, 1), jnp.float32),
    ),
)


def _final_body(acc_ref, h2_ref, es2_ref, ed2_ref, b2_ref, out_ref):
    acc = acc_ref[0] + acc_ref[1]
    ew = es2_ref[...] + ed2_ref[...]
    ew = jnp.where(ew > 0.0, ew, ew * jnp.float32(0.2))
    wself = jnp.exp(ew)
    num = acc + wself * h2_ref[...]
    den = num[:, _NC:_NC + 1]
    o = num / den + b2_ref[...]
    o7 = o[:, 0:_NC]
    m = jnp.max(o7, axis=1, keepdims=True)
    z = o7 - m
    lse = jnp.log(jnp.sum(jnp.exp(z), axis=1, keepdims=True))
    out_ref[...] = z - lse


_final = pl.pallas_call(
    _final_body,
    out_shape=jax.ShapeDtypeStruct((_N, _NC), jnp.float32),
)


def kernel(x, edge_index, W1, a_src1, a_dst1, b1, W2, a_src2, a_dst2, b2):
    sd = edge_index.reshape(2, _NW, _NCHUNK, _CHUNK)
    sd = jnp.pad(sd, ((0, 0), (0, 0), (0, _CROWS - _NCHUNK), (0, 0)))
    src = sd[0].reshape(_NW * _CROWS, _CHUNK)
    dst = sd[1].reshape(_NW * _CROWS, _CHUNK)

    h1, es1, ed1 = _dense1(x, W1, a_src1.reshape(1, _DH),
                           a_dst1.reshape(1, _DH))

    acc1 = _edge_pass1(src, dst, es1.reshape(_N), ed1.reshape(_N), h1)
    acc1 = acc1.reshape(_NCORE, _NP, _DACC1)[:, :_N, :]

    pad = _DH - _NC
    W2p = jnp.pad(W2, ((0, 0), (0, pad)))
    asrc2p = jnp.pad(a_src2, (0, pad)).reshape(1, _DH)
    adst2p = jnp.pad(a_dst2, (0, pad)).reshape(1, _DH)
    b2p = jnp.pad(b2, (0, pad)).reshape(1, _DH)

    h2, es2, ed2 = _combine1(acc1, h1, es1, ed1, b1.reshape(1, _DH),
                             W2p, asrc2p, adst2p)

    acc2 = _edge_pass2(src, dst, es2.reshape(_N), ed2.reshape(_N), h2)
    acc2 = acc2.reshape(_NCORE, _NP, _DACC2)[:, :_N, :]

    return _final(acc2, h2, es2, ed2, b2p)
